# baseline (speedup 1.0000x reference)
import jax
import jax.numpy as jnp
from jax import lax
from jax.experimental import pallas as pl
from jax.experimental.pallas import tpu as pltpu
from jax.experimental.pallas import tpu_sc as plsc

_TOKENS = 8192
_CHANNELS = 4096
_NEURONS = 2048
_ORDERS = 4
_NC = 2
_NS = 16
_NW = _NC * _NS

_TOK_SC = 4608
_TOK_TC = _TOKENS - _TOK_SC

_ROWS = _TOK_SC // _NW
_CHUNK = 8
_NCHUNK = _ROWS // _CHUNK
_LANES = 16
_JBLK = _NEURONS // _LANES

_BT = 256
_BC = 512


def _sc_body(x_hbm, ord_hbm, out_hbm, ord_v,
             in0_v, in1_v, out0_v, out1_v,
             isem0, isem1, osem0, osem1):
    cid = lax.axis_index("c")
    sid = lax.axis_index("s")
    wid = sid * _NC + cid
    row0 = wid * _ROWS

    pltpu.sync_copy(ord_hbm, ord_v)

    def in_src(ci):
        return x_hbm.at[pl.ds(row0 + ci * _CHUNK, _CHUNK)]

    def out_dst(ci):
        return out_hbm.at[pl.ds(row0 + ci * _CHUNK, _CHUNK)]

    def compute(in_v, out_v):
        @plsc.parallel_loop(0, _JBLK, unroll=1)
        def jb_body(jb):
            j16 = jb * _LANES
            idx = [ord_v[i, pl.ds(j16, _LANES)] for i in range(_ORDERS)]
            for r in range(_CHUNK):
                rvec = jnp.full((_LANES,), r, jnp.int32)
                acc = plsc.load_gather(in_v, [rvec, idx[0]])
                for i in range(1, _ORDERS):
                    acc = acc + plsc.load_gather(in_v, [rvec, idx[i]])
                out_v[r, pl.ds(j16, _LANES)] = acc * 0.25

    pltpu.async_copy(in_src(0), in0_v, isem0)

    def chunk_pair(k, _):
        ci0 = 2 * k
        ci1 = ci0 + 1

        pltpu.async_copy(in_src(ci1), in1_v, isem1)
        pltpu.make_async_copy(in_src(ci0), in0_v, isem0).wait()

        @pl.when(k > 0)
        def _():
            pltpu.make_async_copy(out0_v, out_dst(ci0), osem0).wait()

        compute(in0_v, out0_v)
        pltpu.async_copy(out0_v, out_dst(ci0), osem0)

        @pl.when(ci0 + 2 < _NCHUNK)
        def _():
            pltpu.async_copy(in_src(ci0 + 2), in0_v, isem0)

        pltpu.make_async_copy(in_src(ci1), in1_v, isem1).wait()

        @pl.when(k > 0)
        def _():
            pltpu.make_async_copy(out1_v, out_dst(ci1), osem1).wait()

        compute(in1_v, out1_v)
        pltpu.async_copy(out1_v, out_dst(ci1), osem1)
        return 0

    lax.fori_loop(0, _NCHUNK // 2, chunk_pair, 0)

    pltpu.make_async_copy(out0_v, out_dst(_NCHUNK - 2), osem0).wait()
    pltpu.make_async_copy(out1_v, out_dst(_NCHUNK - 1), osem1).wait()


def _sc_call(x, orders):
    mesh = plsc.VectorSubcoreMesh(
        core_axis_name="c", subcore_axis_name="s",
        num_cores=_NC, num_subcores=_NS)
    f = pl.kernel(
        _sc_body,
        out_type=jax.ShapeDtypeStruct((_TOKENS, _NEURONS), jnp.float32),
        mesh=mesh,
        scratch_types=[
            pltpu.VMEM((_ORDERS, _NEURONS), jnp.int32),
            pltpu.VMEM((_CHUNK, _CHANNELS), jnp.float32),
            pltpu.VMEM((_CHUNK, _CHANNELS), jnp.float32),
            pltpu.VMEM((_CHUNK, _NEURONS), jnp.float32),
            pltpu.VMEM((_CHUNK, _NEURONS), jnp.float32),
            pltpu.SemaphoreType.DMA,
            pltpu.SemaphoreType.DMA,
            pltpu.SemaphoreType.DMA,
            pltpu.SemaphoreType.DMA,
        ],
        compiler_params=pltpu.CompilerParams(needs_layout_passes=False),
    )
    return f(x, orders)


def _s_build_body(ord_ref, s_ref):
    c0 = pl.program_id(0) * _BC
    cc = lax.broadcasted_iota(jnp.int32, (_BC, _NEURONS), 0) + c0
    acc = jnp.zeros((_BC, _NEURONS), jnp.float32)
    for i in range(_ORDERS):
        acc += jnp.where(cc == ord_ref[i, :][None, :], 0.25, 0.0)
    s_ref[...] = acc.astype(jnp.bfloat16)


def _s_build(orders):
    return pl.pallas_call(
        _s_build_body,
        grid=(_CHANNELS // _BC,),
        in_specs=[pl.BlockSpec(memory_space=pltpu.VMEM)],
        out_specs=pl.BlockSpec((_BC, _NEURONS), lambda i: (i, 0)),
        out_shape=jax.ShapeDtypeStruct((_CHANNELS, _NEURONS), jnp.bfloat16),
    )(orders)


def _mm_body(x_ref, s_ref, o_ref):
    xb = x_ref[...].astype(jnp.bfloat16)
    o_ref[...] = jnp.dot(xb, s_ref[...],
                         preferred_element_type=jnp.float32)


def _tc_matmul(x, s):
    return pl.pallas_call(
        _mm_body,
        grid=(_TOK_TC // _BT,),
        in_specs=[
            pl.BlockSpec((_BT, _CHANNELS),
                         lambda i: (i + _TOK_SC // _BT, 0)),
            pl.BlockSpec((_CHANNELS, _NEURONS), lambda i: (0, 0)),
        ],
        out_specs=pl.BlockSpec((_BT, _NEURONS), lambda i: (i, 0)),
        out_shape=jax.ShapeDtypeStruct((_TOK_TC, _NEURONS), jnp.float32),
    )(x, s)


def kernel(x, orders):
    sc_out = _sc_call(x, orders)
    s = _s_build(orders)
    tc_out = _tc_matmul(x, s)
    return lax.dynamic_update_slice(sc_out, tc_out, (_TOK_SC, 0))


# speedup vs baseline: 1.0787x; 1.0787x over previous
import jax
import jax.numpy as jnp
from jax import lax
from jax.experimental import pallas as pl
from jax.experimental.pallas import tpu as pltpu
from jax.experimental.pallas import tpu_sc as plsc

_TOKENS = 8192
_CHANNELS = 4096
_NEURONS = 2048
_ORDERS = 4
_NC = 2
_NS = 16
_NW = _NC * _NS

_TOK_SC = 5120
_TOK_TC = _TOKENS - _TOK_SC

_ROWS = _TOK_SC // _NW
_CHUNK = 8
_NCHUNK = _ROWS // _CHUNK
_LANES = 16
_JBLK = _NEURONS // _LANES

_BT = 256
_BC = 512


def _sc_body(x_hbm, ord_hbm, out_hbm, ord_v,
             in0_v, in1_v, out0_v, out1_v,
             isem0, isem1, osem0, osem1):
    cid = lax.axis_index("c")
    sid = lax.axis_index("s")
    wid = sid * _NC + cid
    row0 = wid * _ROWS

    pltpu.sync_copy(ord_hbm, ord_v)

    def in_src(ci):
        return x_hbm.at[pl.ds(row0 + ci * _CHUNK, _CHUNK)]

    def out_dst(ci):
        return out_hbm.at[pl.ds(row0 + ci * _CHUNK, _CHUNK)]

    def compute(in_v, out_v):
        @plsc.parallel_loop(0, _JBLK, unroll=1)
        def jb_body(jb):
            j16 = jb * _LANES
            idx = [ord_v[i, pl.ds(j16, _LANES)] for i in range(_ORDERS)]
            for r in range(_CHUNK):
                rvec = jnp.full((_LANES,), r, jnp.int32)
                acc = plsc.load_gather(in_v, [rvec, idx[0]])
                for i in range(1, _ORDERS):
                    acc = acc + plsc.load_gather(in_v, [rvec, idx[i]])
                out_v[r, pl.ds(j16, _LANES)] = acc * 0.25

    pltpu.async_copy(in_src(0), in0_v, isem0)

    def chunk_pair(k, _):
        ci0 = 2 * k
        ci1 = ci0 + 1

        pltpu.async_copy(in_src(ci1), in1_v, isem1)
        pltpu.make_async_copy(in_src(ci0), in0_v, isem0).wait()

        @pl.when(k > 0)
        def _():
            pltpu.make_async_copy(out0_v, out_dst(ci0), osem0).wait()

        compute(in0_v, out0_v)
        pltpu.async_copy(out0_v, out_dst(ci0), osem0)

        @pl.when(ci0 + 2 < _NCHUNK)
        def _():
            pltpu.async_copy(in_src(ci0 + 2), in0_v, isem0)

        pltpu.make_async_copy(in_src(ci1), in1_v, isem1).wait()

        @pl.when(k > 0)
        def _():
            pltpu.make_async_copy(out1_v, out_dst(ci1), osem1).wait()

        compute(in1_v, out1_v)
        pltpu.async_copy(out1_v, out_dst(ci1), osem1)
        return 0

    lax.fori_loop(0, _NCHUNK // 2, chunk_pair, 0)

    pltpu.make_async_copy(out0_v, out_dst(_NCHUNK - 2), osem0).wait()
    pltpu.make_async_copy(out1_v, out_dst(_NCHUNK - 1), osem1).wait()


def _sc_call(x, orders):
    mesh = plsc.VectorSubcoreMesh(
        core_axis_name="c", subcore_axis_name="s",
        num_cores=_NC, num_subcores=_NS)
    f = pl.kernel(
        _sc_body,
        out_type=jax.ShapeDtypeStruct((_TOKENS, _NEURONS), jnp.float32),
        mesh=mesh,
        scratch_types=[
            pltpu.VMEM((_ORDERS, _NEURONS), jnp.int32),
            pltpu.VMEM((_CHUNK, _CHANNELS), jnp.float32),
            pltpu.VMEM((_CHUNK, _CHANNELS), jnp.float32),
            pltpu.VMEM((_CHUNK, _NEURONS), jnp.float32),
            pltpu.VMEM((_CHUNK, _NEURONS), jnp.float32),
            pltpu.SemaphoreType.DMA,
            pltpu.SemaphoreType.DMA,
            pltpu.SemaphoreType.DMA,
            pltpu.SemaphoreType.DMA,
        ],
        compiler_params=pltpu.CompilerParams(needs_layout_passes=False),
    )
    return f(x, orders)


def _s_build_body(ord_ref, s_ref):
    c0 = pl.program_id(0) * _BC
    cc = lax.broadcasted_iota(jnp.int32, (_BC, _NEURONS), 0) + c0
    acc = jnp.zeros((_BC, _NEURONS), jnp.float32)
    for i in range(_ORDERS):
        acc += jnp.where(cc == ord_ref[i, :][None, :], 0.25, 0.0)
    s_ref[...] = acc.astype(jnp.bfloat16)


def _s_build(orders):
    return pl.pallas_call(
        _s_build_body,
        grid=(_CHANNELS // _BC,),
        in_specs=[pl.BlockSpec(memory_space=pltpu.VMEM)],
        out_specs=pl.BlockSpec((_BC, _NEURONS), lambda i: (i, 0)),
        out_shape=jax.ShapeDtypeStruct((_CHANNELS, _NEURONS), jnp.bfloat16),
    )(orders)


def _mm_body(x_ref, s_ref, o_ref):
    xb = x_ref[...].astype(jnp.bfloat16)
    o_ref[...] = jnp.dot(xb, s_ref[...],
                         preferred_element_type=jnp.float32)


def _tc_matmul(x, s):
    return pl.pallas_call(
        _mm_body,
        grid=(_TOK_TC // _BT,),
        in_specs=[
            pl.BlockSpec((_BT, _CHANNELS),
                         lambda i: (i + _TOK_SC // _BT, 0)),
            pl.BlockSpec((_CHANNELS, _NEURONS), lambda i: (0, 0)),
        ],
        out_specs=pl.BlockSpec((_BT, _NEURONS), lambda i: (i, 0)),
        out_shape=jax.ShapeDtypeStruct((_TOK_TC, _NEURONS), jnp.float32),
    )(x, s)


def _merge_body(sc_ref, tc_ref, o_ref):
    o_ref[...] = tc_ref[...]


def _merge(sc_out, tc_out):
    return pl.pallas_call(
        _merge_body,
        grid=(_TOK_TC // _BT,),
        in_specs=[
            pl.BlockSpec((8, _NEURONS), lambda i: (0, 0)),
            pl.BlockSpec((_BT, _NEURONS), lambda i: (i, 0)),
        ],
        out_specs=pl.BlockSpec((_BT, _NEURONS),
                               lambda i: (i + _TOK_SC // _BT, 0)),
        out_shape=jax.ShapeDtypeStruct((_TOKENS, _NEURONS), jnp.float32),
        input_output_aliases={0: 0},
    )(sc_out, tc_out)


def kernel(x, orders):
    sc_out = _sc_call(x, orders)
    s = _s_build(orders)
    tc_out = _tc_matmul(x, s)
    return _merge(sc_out, tc_out)


# speedup vs baseline: 1.0897x; 1.0102x over previous
import jax
import jax.numpy as jnp
from jax import lax
from jax.experimental import pallas as pl
from jax.experimental.pallas import tpu as pltpu
from jax.experimental.pallas import tpu_sc as plsc

_TOKENS = 8192
_CHANNELS = 4096
_NEURONS = 2048
_ORDERS = 4
_NC = 2
_NS = 16
_NW = _NC * _NS

_TOK_SC = 5120
_TOK_TC = _TOKENS - _TOK_SC

_ROWS = _TOK_SC // _NW
_CHUNK = 8
_NCHUNK = _ROWS // _CHUNK
_LANES = 16
_JBLK = _NEURONS // _LANES

_BT = 256
_BC = 512


def _sc_body(x_hbm, ord_hbm, out_hbm, ord_v,
             in0_v, in1_v, out0_v, out1_v,
             isem0, isem1, osem0, osem1):
    cid = lax.axis_index("c")
    sid = lax.axis_index("s")
    wid = sid * _NC + cid
    row0 = wid * _ROWS

    pltpu.sync_copy(ord_hbm, ord_v)

    def in_src(ci):
        return x_hbm.at[pl.ds(row0 + ci * _CHUNK, _CHUNK)]

    def out_dst(ci):
        return out_hbm.at[pl.ds(row0 + ci * _CHUNK, _CHUNK)]

    def compute(in_v, out_v):
        @plsc.parallel_loop(0, _JBLK, unroll=1)
        def jb_body(jb):
            j16 = jb * _LANES
            idx = [ord_v[i, pl.ds(j16, _LANES)] for i in range(_ORDERS)]
            for r in range(_CHUNK):
                rvec = jnp.full((_LANES,), r, jnp.int32)
                acc = plsc.load_gather(in_v, [rvec, idx[0]])
                for i in range(1, _ORDERS):
                    acc = acc + plsc.load_gather(in_v, [rvec, idx[i]])
                out_v[r, pl.ds(j16, _LANES)] = acc * 0.25

    pltpu.async_copy(in_src(0), in0_v, isem0)

    def chunk_pair(k, _):
        ci0 = 2 * k
        ci1 = ci0 + 1

        pltpu.async_copy(in_src(ci1), in1_v, isem1)
        pltpu.make_async_copy(in_src(ci0), in0_v, isem0).wait()

        @pl.when(k > 0)
        def _():
            pltpu.make_async_copy(out0_v, out_dst(ci0), osem0).wait()

        compute(in0_v, out0_v)
        pltpu.async_copy(out0_v, out_dst(ci0), osem0)

        @pl.when(ci0 + 2 < _NCHUNK)
        def _():
            pltpu.async_copy(in_src(ci0 + 2), in0_v, isem0)

        pltpu.make_async_copy(in_src(ci1), in1_v, isem1).wait()

        @pl.when(k > 0)
        def _():
            pltpu.make_async_copy(out1_v, out_dst(ci1), osem1).wait()

        compute(in1_v, out1_v)
        pltpu.async_copy(out1_v, out_dst(ci1), osem1)
        return 0

    lax.fori_loop(0, _NCHUNK // 2, chunk_pair, 0)

    pltpu.make_async_copy(out0_v, out_dst(_NCHUNK - 2), osem0).wait()
    pltpu.make_async_copy(out1_v, out_dst(_NCHUNK - 1), osem1).wait()


def _sc_call(x, orders):
    mesh = plsc.VectorSubcoreMesh(
        core_axis_name="c", subcore_axis_name="s",
        num_cores=_NC, num_subcores=_NS)
    f = pl.kernel(
        _sc_body,
        out_type=jax.ShapeDtypeStruct((_TOKENS, _NEURONS), jnp.float32),
        mesh=mesh,
        scratch_types=[
            pltpu.VMEM((_ORDERS, _NEURONS), jnp.int32),
            pltpu.VMEM((_CHUNK, _CHANNELS), jnp.float32),
            pltpu.VMEM((_CHUNK, _CHANNELS), jnp.float32),
            pltpu.VMEM((_CHUNK, _NEURONS), jnp.float32),
            pltpu.VMEM((_CHUNK, _NEURONS), jnp.float32),
            pltpu.SemaphoreType.DMA,
            pltpu.SemaphoreType.DMA,
            pltpu.SemaphoreType.DMA,
            pltpu.SemaphoreType.DMA,
        ],
        compiler_params=pltpu.CompilerParams(needs_layout_passes=False),
    )
    return f(x, orders)


def _s_build_body(ord_ref, s_ref):
    c0 = pl.program_id(0) * _BC
    cc = lax.broadcasted_iota(jnp.int32, (_BC, _NEURONS), 0) + c0
    acc = jnp.zeros((_BC, _NEURONS), jnp.float32)
    for i in range(_ORDERS):
        acc += jnp.where(cc == ord_ref[i, :][None, :], 0.25, 0.0)
    s_ref[...] = acc.astype(jnp.bfloat16)


def _s_build(orders):
    return pl.pallas_call(
        _s_build_body,
        grid=(_CHANNELS // _BC,),
        in_specs=[pl.BlockSpec(memory_space=pltpu.VMEM)],
        out_specs=pl.BlockSpec((_BC, _NEURONS), lambda i: (i, 0)),
        out_shape=jax.ShapeDtypeStruct((_CHANNELS, _NEURONS), jnp.bfloat16),
    )(orders)


def _mm_body(x_ref, s_ref, o_ref):
    xb = x_ref[...].astype(jnp.bfloat16)
    o_ref[...] = jnp.dot(xb, s_ref[...],
                         preferred_element_type=jnp.float32)


def _tc_matmul(x, s):
    return pl.pallas_call(
        _mm_body,
        grid=(_TOK_TC // _BT,),
        in_specs=[
            pl.BlockSpec((_BT, _CHANNELS),
                         lambda i: (i + _TOK_SC // _BT, 0)),
            pl.BlockSpec((_CHANNELS, _NEURONS), lambda i: (0, 0)),
        ],
        out_specs=pl.BlockSpec((_BT, _NEURONS), lambda i: (i, 0)),
        out_shape=jax.ShapeDtypeStruct((_TOK_TC, _NEURONS), jnp.float32),
    )(x, s)


def kernel(x, orders):
    sc_out = _sc_call(x, orders)
    s = _s_build(orders)
    tc_out = _tc_matmul(x, s)
    return lax.dynamic_update_slice(sc_out, tc_out, (_TOK_SC, 0))
